# baseline (speedup 1.0000x reference)
import functools

import jax
import jax.numpy as jnp
from jax import lax
from jax.experimental import pallas as pl
from jax.experimental.pallas import tpu as pltpu
from jax.experimental.pallas import tpu_sc as plsc

NC = 2
NS = 16
NW = NC * NS
L = 16

D_MODEL = 64
D_PAD = 128
SCALE = 8.0

BB = 128
NBUF = 4
AHEAD = 2


def _make_emb_kernel(n_rows: int, n_cols: int):
    kb_n = n_rows // BB
    nchunks = n_cols * kb_n
    assert n_rows % BB == 0 and nchunks % NW == 0
    nch = nchunks // NW
    assert nch % NBUF == 0
    ngrp = nch // NBUF

    mesh = plsc.VectorSubcoreMesh(
        core_axis_name="c", subcore_axis_name="s",
        num_cores=NC, num_subcores=NS,
    )

    scratch = [pltpu.VMEM((nch, BB), jnp.int32)]
    scratch += [pltpu.VMEM((BB, D_PAD), jnp.float32) for _ in range(NBUF)]
    scratch += [pltpu.VMEM((D_MODEL, BB), jnp.float32) for _ in range(NBUF)]
    scratch += [pltpu.SemaphoreType.DMA for _ in range(2 * NBUF)]

    @functools.partial(
        pl.kernel,
        out_type=jax.ShapeDtypeStruct((n_cols * D_MODEL, n_rows), jnp.float32),
        mesh=mesh,
        scratch_types=scratch,
        compiler_params=pltpu.CompilerParams(
            use_tc_tiling_on_sc=True, needs_layout_passes=False),
    )
    def emb(idx_hbm, table_hbm, out_hbm, idx_v, *rest):
        rows = rest[:NBUF]
        tbuf = rest[NBUF:2 * NBUF]
        sem_in = rest[2 * NBUF:3 * NBUF]
        sem_out = rest[3 * NBUF:]

        wid = lax.axis_index("s") * NC + lax.axis_index("c")
        cid0 = wid * nch

        pltpu.sync_copy(idx_hbm.at[wid], idx_v)

        lanes = lax.iota(jnp.int32, L)

        def fire_gather(f, bf):
            pltpu.async_copy(table_hbm.at[idx_v.at[f]], rows[bf], sem_in[bf])

        def drain_gather(bf):
            pltpu.make_async_copy(
                table_hbm.at[pl.ds(0, BB)], rows[bf], sem_in[bf]).wait()

        def drain_store(bf):
            pltpu.make_async_copy(
                tbuf[bf], out_hbm.at[pl.ds(0, D_MODEL), pl.ds(0, BB)],
                sem_out[bf]).wait()

        for b in range(AHEAD):
            fire_gather(b, b)

        def group(i, carry):
            for b in range(NBUF):
                g = i * NBUF + b
                cid = cid0 + g
                s = cid // kb_n
                kb = cid - s * kb_n
                r = rows[b]
                t = tbuf[b]

                drain_gather(b)

                @pl.when(i > 0)
                def _drain(b=b):
                    drain_store(b)

                def trans_col(c, carry2, r=r, t=t):
                    col = jnp.full((L,), c, jnp.int32)
                    for j0 in range(BB // L):
                        v = plsc.load_gather(r, [lanes + (j0 * L), col])
                        t[c, pl.ds(j0 * L, L)] = v * SCALE
                    return carry2
                lax.fori_loop(0, D_MODEL, trans_col, 0, unroll=2)

                pltpu.async_copy(
                    t,
                    out_hbm.at[pl.ds(s * D_MODEL, D_MODEL),
                               pl.ds(kb * BB, BB)],
                    sem_out[b])

                @pl.when(g + AHEAD < nch)
                def _fire(g=g, b=b):
                    fire_gather(g + AHEAD, (b + AHEAD) % NBUF)
            return carry

        lax.fori_loop(0, ngrp, group, 0)

        for b in range(NBUF):
            drain_store(b)

    return emb


@functools.lru_cache(maxsize=None)
def _get_emb(n_rows: int, n_cols: int):
    return _make_emb_kernel(n_rows, n_cols)


def kernel(x, table):
    n_rows, n_cols = x.shape
    idx = x.T.astype(jnp.int32).reshape(NW, (n_cols * n_rows) // (NW * BB), BB)
    tablep = jnp.pad(table, ((0, 0), (0, D_PAD - D_MODEL)))
    out = _get_emb(n_rows, n_cols)(idx, tablep)
    return out.reshape(n_cols, D_MODEL, n_rows).transpose(2, 0, 1)


# speedup vs baseline: 1.0564x; 1.0564x over previous
import functools

import jax
import jax.numpy as jnp
from jax import lax
from jax.experimental import pallas as pl
from jax.experimental.pallas import tpu as pltpu
from jax.experimental.pallas import tpu_sc as plsc

NC = 2
NS = 16
NW = NC * NS
L = 16

D_MODEL = 64
D_PAD = 128
SCALE = 8.0

BB = 128
NBUF = 4
AHEAD = 2


def _make_emb_kernel(n_rows: int, n_cols: int):
    kb_n = n_rows // BB
    nchunks = n_cols * kb_n
    assert n_rows % BB == 0 and nchunks % NW == 0
    nch = nchunks // NW
    assert nch % NBUF == 0
    ngrp = nch // NBUF

    mesh = plsc.VectorSubcoreMesh(
        core_axis_name="c", subcore_axis_name="s",
        num_cores=NC, num_subcores=NS,
    )

    scratch = [pltpu.VMEM((nch, BB), jnp.int32)]
    scratch += [pltpu.VMEM((BB, D_PAD), jnp.float32) for _ in range(NBUF)]
    scratch += [pltpu.VMEM((D_MODEL, BB), jnp.float32) for _ in range(NBUF)]
    scratch += [pltpu.SemaphoreType.DMA for _ in range(2 * NBUF)]

    @functools.partial(
        pl.kernel,
        out_type=jax.ShapeDtypeStruct((n_cols * D_MODEL, n_rows), jnp.float32),
        mesh=mesh,
        scratch_types=scratch,
        compiler_params=pltpu.CompilerParams(
            use_tc_tiling_on_sc=True, needs_layout_passes=False),
    )
    def emb(idx_hbm, table_hbm, out_hbm, idx_v, *rest):
        rows = rest[:NBUF]
        tbuf = rest[NBUF:2 * NBUF]
        sem_in = rest[2 * NBUF:3 * NBUF]
        sem_out = rest[3 * NBUF:]

        wid = lax.axis_index("s") * NC + lax.axis_index("c")
        cid0 = wid * nch

        pltpu.sync_copy(idx_hbm.at[wid], idx_v)

        rvecs = [lax.iota(jnp.int32, L) + (j0 * L) for j0 in range(BB // L)]

        def fire_gather(f, bf):
            pltpu.async_copy(table_hbm.at[idx_v.at[f]], rows[bf], sem_in[bf])

        def drain_gather(bf):
            pltpu.make_async_copy(
                table_hbm.at[pl.ds(0, BB)], rows[bf], sem_in[bf]).wait()

        def drain_store(bf):
            pltpu.make_async_copy(
                tbuf[bf], out_hbm.at[pl.ds(0, D_MODEL), pl.ds(0, BB)],
                sem_out[bf]).wait()

        for b in range(AHEAD):
            fire_gather(b, b)

        def group(i, carry):
            for b in range(NBUF):
                g = i * NBUF + b
                cid = cid0 + g
                s = cid // kb_n
                kb = cid - s * kb_n
                r = rows[b]
                t = tbuf[b]

                drain_gather(b)

                @pl.when(i > 0)
                def _drain(b=b):
                    drain_store(b)

                def trans_col(c, carry2, r=r, t=t):
                    col = jnp.full((L,), c, jnp.int32)
                    for j0 in range(BB // L):
                        v = plsc.load_gather(r, [rvecs[j0], col])
                        t[c, pl.ds(j0 * L, L)] = v
                    return carry2
                lax.fori_loop(0, D_MODEL, trans_col, 0, unroll=8)

                pltpu.async_copy(
                    t,
                    out_hbm.at[pl.ds(s * D_MODEL, D_MODEL),
                               pl.ds(kb * BB, BB)],
                    sem_out[b])

                @pl.when(g + AHEAD < nch)
                def _fire(g=g, b=b):
                    fire_gather(g + AHEAD, (b + AHEAD) % NBUF)
            return carry

        lax.fori_loop(0, ngrp, group, 0)

        for b in range(NBUF):
            drain_store(b)

    return emb


@functools.lru_cache(maxsize=None)
def _get_emb(n_rows: int, n_cols: int):
    return _make_emb_kernel(n_rows, n_cols)


def kernel(x, table):
    n_rows, n_cols = x.shape
    idx = x.T.astype(jnp.int32).reshape(NW, (n_cols * n_rows) // (NW * BB), BB)
    tablep = jnp.pad(table * jnp.float32(SCALE), ((0, 0), (0, D_PAD - D_MODEL)))
    out = _get_emb(n_rows, n_cols)(idx, tablep)
    return out.reshape(n_cols, D_MODEL, n_rows).transpose(2, 0, 1)


# speedup vs baseline: 2.6696x; 2.5270x over previous
import functools

import jax
import jax.numpy as jnp
from jax import lax
from jax.experimental import pallas as pl
from jax.experimental.pallas import tpu as pltpu
from jax.experimental.pallas import tpu_sc as plsc

NC = 2
NS = 16
NW = NC * NS
L = 16

D_MODEL = 64
D_PAD = 128
SCALE = 8.0

BB = 128
NBUF = 4
AHEAD = 2


def _make_emb_kernel(n_rows: int, n_cols: int):
    kb_n = n_rows // BB
    nchunks = n_cols * kb_n
    assert n_rows % BB == 0 and nchunks % NW == 0
    nch = nchunks // NW
    assert nch % NBUF == 0
    ngrp = nch // NBUF

    mesh = plsc.VectorSubcoreMesh(
        core_axis_name="c", subcore_axis_name="s",
        num_cores=NC, num_subcores=NS,
    )

    scratch = [pltpu.VMEM((nch, BB), jnp.int32)]
    scratch += [pltpu.VMEM((BB, D_PAD), jnp.float32) for _ in range(NBUF)]
    scratch += [pltpu.VMEM((D_MODEL, BB), jnp.float32) for _ in range(NBUF)]
    scratch += [pltpu.SemaphoreType.DMA for _ in range(2 * NBUF)]

    @functools.partial(
        pl.kernel,
        out_type=jax.ShapeDtypeStruct((n_cols * D_MODEL, n_rows), jnp.float32),
        mesh=mesh,
        scratch_types=scratch,
        compiler_params=pltpu.CompilerParams(
            use_tc_tiling_on_sc=True, needs_layout_passes=False),
    )
    def emb(idx_hbm, table_hbm, out_hbm, idx_v, *rest):
        rows = rest[:NBUF]
        tbuf = rest[NBUF:2 * NBUF]
        sem_in = rest[2 * NBUF:3 * NBUF]
        sem_out = rest[3 * NBUF:]

        wid = lax.axis_index("s") * NC + lax.axis_index("c")
        cid0 = wid * nch

        pltpu.sync_copy(idx_hbm.at[wid], idx_v)

        lanes = lax.iota(jnp.int32, L)
        rvecs = [lanes + (j0 * L) for j0 in range(BB // L)]


        def fire_gather(f, bf):
            pltpu.async_copy(table_hbm.at[idx_v.at[f]], rows[bf], sem_in[bf])

        def drain_gather(bf):
            pltpu.make_async_copy(
                table_hbm.at[pl.ds(0, BB)], rows[bf], sem_in[bf]).wait()

        def drain_store(bf):
            pltpu.make_async_copy(
                tbuf[bf], out_hbm.at[pl.ds(0, D_MODEL), pl.ds(0, BB)],
                sem_out[bf]).wait()

        for b in range(AHEAD):
            fire_gather(b, b)

        def group(i, carry):
            for b in range(NBUF):
                g = i * NBUF + b
                cid = cid0 + g
                s = cid // kb_n
                kb = cid - s * kb_n
                r = rows[b]
                t = tbuf[b]

                drain_gather(b)

                @pl.when(i > 0)
                def _drain(b=b):
                    drain_store(b)

                def trans_c0(c0, carry2, r=r, t=t):
                    cbase = c0 * L
                    def diag(d, carry3, r=r, t=t, cbase=cbase):
                        rot = jnp.where(lanes < L - d, lanes + d,
                                        lanes + d - L)
                        col = rot + cbase
                        for j0 in range(BB // L):
                            v = plsc.load_gather(r, [rvecs[j0], col])
                            plsc.store_scatter(t, [col, rvecs[j0]], v)
                        return carry3
                    lax.fori_loop(0, L, diag, 0)
                    return carry2
                lax.fori_loop(0, D_MODEL // L, trans_c0, 0)

                pltpu.async_copy(
                    t,
                    out_hbm.at[pl.ds(s * D_MODEL, D_MODEL),
                               pl.ds(kb * BB, BB)],
                    sem_out[b])

                @pl.when(g + AHEAD < nch)
                def _fire(g=g, b=b):
                    fire_gather(g + AHEAD, (b + AHEAD) % NBUF)
            return carry

        lax.fori_loop(0, ngrp, group, 0)

        for b in range(NBUF):
            drain_store(b)

    return emb


@functools.lru_cache(maxsize=None)
def _get_emb(n_rows: int, n_cols: int):
    return _make_emb_kernel(n_rows, n_cols)


def kernel(x, table):
    n_rows, n_cols = x.shape
    idx = x.T.astype(jnp.int32).reshape(NW, (n_cols * n_rows) // (NW * BB), BB)
    tablep = jnp.pad(table * jnp.float32(SCALE), ((0, 0), (0, D_PAD - D_MODEL)))
    out = _get_emb(n_rows, n_cols)(idx, tablep)
    return out.reshape(n_cols, D_MODEL, n_rows).transpose(2, 0, 1)


# speedup vs baseline: 2.7429x; 1.0275x over previous
import functools

import jax
import jax.numpy as jnp
from jax import lax
from jax.experimental import pallas as pl
from jax.experimental.pallas import tpu as pltpu
from jax.experimental.pallas import tpu_sc as plsc

NC = 2
NS = 16
NW = NC * NS
L = 16

D_MODEL = 64
D_PAD = 128
SCALE = 8.0

BB = 128
NBUF = 4
AHEAD = 2


def _make_emb_kernel(n_rows: int, n_cols: int):
    kb_n = n_rows // BB
    nchunks = n_cols * kb_n
    assert n_rows % BB == 0 and nchunks % NW == 0
    nch = nchunks // NW
    assert nch % NBUF == 0
    ngrp = nch // NBUF

    mesh = plsc.VectorSubcoreMesh(
        core_axis_name="c", subcore_axis_name="s",
        num_cores=NC, num_subcores=NS,
    )

    scratch = [pltpu.VMEM((nch, BB), jnp.int32)]
    scratch += [pltpu.VMEM((BB, D_PAD), jnp.float32) for _ in range(NBUF)]
    scratch += [pltpu.VMEM((D_MODEL, BB), jnp.float32) for _ in range(NBUF)]
    scratch += [pltpu.SemaphoreType.DMA for _ in range(2 * NBUF)]

    @functools.partial(
        pl.kernel,
        out_type=jax.ShapeDtypeStruct((n_cols * D_MODEL, n_rows), jnp.float32),
        mesh=mesh,
        scratch_types=scratch,
        compiler_params=pltpu.CompilerParams(
            use_tc_tiling_on_sc=True, needs_layout_passes=False),
    )
    def emb(idx_hbm, table_hbm, out_hbm, idx_v, *rest):
        rows = rest[:NBUF]
        tbuf = rest[NBUF:2 * NBUF]
        sem_in = rest[2 * NBUF:3 * NBUF]
        sem_out = rest[3 * NBUF:]

        wid = lax.axis_index("s") * NC + lax.axis_index("c")
        cid0 = wid * nch

        pltpu.sync_copy(idx_hbm.at[wid], idx_v)

        lanes = lax.iota(jnp.int32, L)
        rvecs = [lanes + (j0 * L) for j0 in range(BB // L)]


        def fire_gather(f, bf):
            pltpu.async_copy(table_hbm.at[idx_v.at[f]], rows[bf], sem_in[bf])

        def drain_gather(bf):
            pltpu.make_async_copy(
                table_hbm.at[pl.ds(0, BB)], rows[bf], sem_in[bf]).wait()

        def drain_store(bf):
            pltpu.make_async_copy(
                tbuf[bf], out_hbm.at[pl.ds(0, D_MODEL), pl.ds(0, BB)],
                sem_out[bf]).wait()

        for b in range(AHEAD):
            fire_gather(b, b)

        def group(i, carry):
            for b in range(NBUF):
                g = i * NBUF + b
                cid = cid0 + g
                s = cid // kb_n
                kb = cid - s * kb_n
                r = rows[b]
                t = tbuf[b]

                drain_gather(b)

                @pl.when(i > 0)
                def _drain(b=b):
                    drain_store(b)

                def diag(d, carry2, r=r, t=t):
                    rot = jnp.where(lanes < L - d, lanes + d, lanes + d - L)
                    for c0 in range(D_MODEL // L):
                        col = rot + (c0 * L)
                        for j0 in range(BB // L):
                            v = plsc.load_gather(r, [rvecs[j0], col])
                            plsc.store_scatter(t, [col, rvecs[j0]], v)
                    return carry2
                lax.fori_loop(0, L, diag, 0)

                pltpu.async_copy(
                    t,
                    out_hbm.at[pl.ds(s * D_MODEL, D_MODEL),
                               pl.ds(kb * BB, BB)],
                    sem_out[b])

                @pl.when(g + AHEAD < nch)
                def _fire(g=g, b=b):
                    fire_gather(g + AHEAD, (b + AHEAD) % NBUF)
            return carry

        lax.fori_loop(0, ngrp, group, 0)

        for b in range(NBUF):
            drain_store(b)

    return emb


@functools.lru_cache(maxsize=None)
def _get_emb(n_rows: int, n_cols: int):
    return _make_emb_kernel(n_rows, n_cols)


def kernel(x, table):
    n_rows, n_cols = x.shape
    idx = x.T.astype(jnp.int32).reshape(NW, (n_cols * n_rows) // (NW * BB), BB)
    tablep = jnp.pad(table * jnp.float32(SCALE), ((0, 0), (0, D_PAD - D_MODEL)))
    out = _get_emb(n_rows, n_cols)(idx, tablep)
    return out.reshape(n_cols, D_MODEL, n_rows).transpose(2, 0, 1)


# speedup vs baseline: 3.5928x; 1.3098x over previous
import functools

import jax
import jax.numpy as jnp
from jax import lax
from jax.experimental import pallas as pl
from jax.experimental.pallas import tpu as pltpu
from jax.experimental.pallas import tpu_sc as plsc

NC = 2
NS = 16
NW = NC * NS
L = 16

D_MODEL = 64
D_PAD = 128
SCALE = 8.0

BB = 128
NBUF = 4
AHEAD = 2


def _make_emb_kernel(n_rows: int, n_cols: int):
    kb_n = n_rows // BB
    nchunks = n_cols * kb_n
    assert n_rows % BB == 0 and nchunks % NW == 0
    nch = nchunks // NW
    assert nch % NBUF == 0
    ngrp = nch // NBUF

    mesh = plsc.VectorSubcoreMesh(
        core_axis_name="c", subcore_axis_name="s",
        num_cores=NC, num_subcores=NS,
    )

    scratch = [pltpu.VMEM((nch, BB), jnp.int32)]
    scratch += [pltpu.VMEM((BB, D_PAD), jnp.float32) for _ in range(NBUF)]
    scratch += [pltpu.VMEM((D_MODEL, BB), jnp.float32) for _ in range(NBUF)]
    scratch += [pltpu.SemaphoreType.DMA for _ in range(2 * NBUF)]

    @functools.partial(
        pl.kernel,
        out_type=jax.ShapeDtypeStruct((n_cols * D_MODEL, n_rows), jnp.float32),
        mesh=mesh,
        scratch_types=scratch,
        compiler_params=pltpu.CompilerParams(
            use_tc_tiling_on_sc=True, needs_layout_passes=False),
    )
    def emb(idx_hbm, table_hbm, out_hbm, idx_v, *rest):
        rows = rest[:NBUF]
        tbuf = rest[NBUF:2 * NBUF]
        sem_in = rest[2 * NBUF:3 * NBUF]
        sem_out = rest[3 * NBUF:]

        wid = lax.axis_index("s") * NC + lax.axis_index("c")
        cid0 = wid * nch

        pltpu.sync_copy(idx_hbm.at[wid], idx_v)

        lanes = lax.iota(jnp.int32, L)
        rvecs = [lanes + (j0 * L) for j0 in range(BB // L)]


        def fire_gather(f, bf):
            pltpu.async_copy(table_hbm.at[idx_v.at[f]], rows[bf], sem_in[bf])

        def drain_gather(bf):
            pltpu.make_async_copy(
                table_hbm.at[pl.ds(0, BB)], rows[bf], sem_in[bf]).wait()

        def drain_store(bf):
            pltpu.make_async_copy(
                tbuf[bf], out_hbm.at[pl.ds(0, D_MODEL), pl.ds(0, BB)],
                sem_out[bf]).wait()

        for b in range(AHEAD):
            fire_gather(b, b)

        def group(i, carry):
            for b in range(NBUF):
                g = i * NBUF + b
                cid = cid0 + g
                s = cid // kb_n
                kb = cid - s * kb_n
                r = rows[b]
                t = tbuf[b]

                drain_gather(b)

                @pl.when(i > 0)
                def _drain(b=b):
                    drain_store(b)

                @plsc.parallel_loop(0, L, unroll=2)
                def diag(d, r=r, t=t):
                    rot = jnp.where(lanes < L - d, lanes + d, lanes + d - L)
                    for c0 in range(D_MODEL // L):
                        col = rot + (c0 * L)
                        for j0 in range(BB // L):
                            v = plsc.load_gather(r, [rvecs[j0], col])
                            plsc.store_scatter(t, [col, rvecs[j0]], v)

                pltpu.async_copy(
                    t,
                    out_hbm.at[pl.ds(s * D_MODEL, D_MODEL),
                               pl.ds(kb * BB, BB)],
                    sem_out[b])

                @pl.when(g + AHEAD < nch)
                def _fire(g=g, b=b):
                    fire_gather(g + AHEAD, (b + AHEAD) % NBUF)
            return carry

        lax.fori_loop(0, ngrp, group, 0)

        for b in range(NBUF):
            drain_store(b)

    return emb


@functools.lru_cache(maxsize=None)
def _get_emb(n_rows: int, n_cols: int):
    return _make_emb_kernel(n_rows, n_cols)


def kernel(x, table):
    n_rows, n_cols = x.shape
    idx = x.T.astype(jnp.int32).reshape(NW, (n_cols * n_rows) // (NW * BB), BB)
    tablep = jnp.pad(table * jnp.float32(SCALE), ((0, 0), (0, D_PAD - D_MODEL)))
    out = _get_emb(n_rows, n_cols)(idx, tablep)
    return out.reshape(n_cols, D_MODEL, n_rows).transpose(2, 0, 1)


# speedup vs baseline: 3.9496x; 1.0993x over previous
import functools

import jax
import jax.numpy as jnp
from jax import lax
from jax.experimental import pallas as pl
from jax.experimental.pallas import tpu as pltpu
from jax.experimental.pallas import tpu_sc as plsc

NC = 2
NS = 16
NW = NC * NS
L = 16

D_MODEL = 64
D_PAD = 128
SCALE = 8.0

BB = 128
NBUF = 4
AHEAD = 2


def _make_emb_kernel(n_rows: int, n_cols: int):
    kb_n = n_rows // BB
    nchunks = n_cols * kb_n
    assert n_rows % BB == 0 and nchunks % NW == 0
    nch = nchunks // NW
    assert nch % NBUF == 0
    ngrp = nch // NBUF

    mesh = plsc.VectorSubcoreMesh(
        core_axis_name="c", subcore_axis_name="s",
        num_cores=NC, num_subcores=NS,
    )

    scratch = [pltpu.VMEM((nch, BB), jnp.int32)]
    scratch += [pltpu.VMEM((BB, D_PAD), jnp.float32) for _ in range(NBUF)]
    scratch += [pltpu.VMEM((D_MODEL, BB), jnp.float32) for _ in range(NBUF)]
    scratch += [pltpu.SemaphoreType.DMA for _ in range(2 * NBUF)]

    @functools.partial(
        pl.kernel,
        out_type=jax.ShapeDtypeStruct((n_cols * D_MODEL, n_rows), jnp.float32),
        mesh=mesh,
        scratch_types=scratch,
        compiler_params=pltpu.CompilerParams(
            use_tc_tiling_on_sc=True, needs_layout_passes=False),
    )
    def emb(idx_hbm, table_hbm, out_hbm, idx_v, *rest):
        rows = rest[:NBUF]
        tbuf = rest[NBUF:2 * NBUF]
        sem_in = rest[2 * NBUF:3 * NBUF]
        sem_out = rest[3 * NBUF:]

        wid = lax.axis_index("s") * NC + lax.axis_index("c")
        cid0 = wid * nch

        pltpu.sync_copy(idx_hbm.at[wid], idx_v)

        lanes = lax.iota(jnp.int32, L)
        rvecs = [lanes + (j0 * L) for j0 in range(BB // L)]


        def fire_gather(f, bf):
            pltpu.async_copy(table_hbm.at[idx_v.at[f]], rows[bf], sem_in[bf])

        def drain_gather(bf):
            pltpu.make_async_copy(
                table_hbm.at[pl.ds(0, BB)], rows[bf], sem_in[bf]).wait()

        def drain_store(bf):
            pltpu.make_async_copy(
                tbuf[bf], out_hbm.at[pl.ds(0, D_MODEL), pl.ds(0, BB)],
                sem_out[bf]).wait()

        for b in range(AHEAD):
            fire_gather(b, b)

        def group(i, carry):
            for b in range(NBUF):
                g = i * NBUF + b
                cid = cid0 + g
                s = cid // kb_n
                kb = cid - s * kb_n
                r = rows[b]
                t = tbuf[b]

                drain_gather(b)

                @pl.when(i > 0)
                def _drain(b=b):
                    drain_store(b)

                @plsc.parallel_loop(0, L, unroll=4)
                def diag(d, r=r, t=t):
                    rot = jnp.where(lanes < L - d, lanes + d, lanes + d - L)
                    for c0 in range(D_MODEL // L):
                        col = rot + (c0 * L)
                        for j0 in range(BB // L):
                            v = plsc.load_gather(r, [rvecs[j0], col])
                            plsc.store_scatter(t, [col, rvecs[j0]], v)

                pltpu.async_copy(
                    t,
                    out_hbm.at[pl.ds(s * D_MODEL, D_MODEL),
                               pl.ds(kb * BB, BB)],
                    sem_out[b])

                @pl.when(g + AHEAD < nch)
                def _fire(g=g, b=b):
                    fire_gather(g + AHEAD, (b + AHEAD) % NBUF)
            return carry

        lax.fori_loop(0, ngrp, group, 0)

        for b in range(NBUF):
            drain_store(b)

    return emb


@functools.lru_cache(maxsize=None)
def _get_emb(n_rows: int, n_cols: int):
    return _make_emb_kernel(n_rows, n_cols)


def kernel(x, table):
    n_rows, n_cols = x.shape
    idx = x.T.astype(jnp.int32).reshape(NW, (n_cols * n_rows) // (NW * BB), BB)
    tablep = jnp.pad(table * jnp.float32(SCALE), ((0, 0), (0, D_PAD - D_MODEL)))
    out = _get_emb(n_rows, n_cols)(idx, tablep)
    return out.reshape(n_cols, D_MODEL, n_rows).transpose(2, 0, 1)


# speedup vs baseline: 3.9562x; 1.0017x over previous
import functools

import jax
import jax.numpy as jnp
from jax import lax
from jax.experimental import pallas as pl
from jax.experimental.pallas import tpu as pltpu
from jax.experimental.pallas import tpu_sc as plsc

NC = 2
NS = 16
NW = NC * NS
L = 16

D_MODEL = 64
D_PAD = 128
SCALE = 8.0

BB = 128
NBUF = 4
AHEAD = 2


def _make_emb_kernel(n_rows: int, n_cols: int):
    kb_n = n_rows // BB
    nchunks = n_cols * kb_n
    assert n_rows % BB == 0 and nchunks % NW == 0
    nch = nchunks // NW
    assert nch % NBUF == 0
    ngrp = nch // NBUF

    mesh = plsc.VectorSubcoreMesh(
        core_axis_name="c", subcore_axis_name="s",
        num_cores=NC, num_subcores=NS,
    )

    scratch = [pltpu.VMEM((nch, BB), jnp.int32)]
    scratch += [pltpu.VMEM((BB, D_PAD), jnp.float32) for _ in range(NBUF)]
    scratch += [pltpu.VMEM((D_MODEL, BB), jnp.float32) for _ in range(NBUF)]
    scratch += [pltpu.SemaphoreType.DMA for _ in range(2 * NBUF)]

    @functools.partial(
        pl.kernel,
        out_type=jax.ShapeDtypeStruct((n_cols * D_MODEL, n_rows), jnp.float32),
        mesh=mesh,
        scratch_types=scratch,
        compiler_params=pltpu.CompilerParams(
            use_tc_tiling_on_sc=True, needs_layout_passes=False),
    )
    def emb(idx_hbm, table_hbm, out_hbm, idx_v, *rest):
        rows = rest[:NBUF]
        tbuf = rest[NBUF:2 * NBUF]
        sem_in = rest[2 * NBUF:3 * NBUF]
        sem_out = rest[3 * NBUF:]

        wid = lax.axis_index("s") * NC + lax.axis_index("c")
        cid0 = wid * nch

        pltpu.sync_copy(idx_hbm.at[wid], idx_v)

        lanes = lax.iota(jnp.int32, L)
        rvecs = [lanes + (j0 * L) for j0 in range(BB // L)]


        def fire_gather(f, bf):
            pltpu.async_copy(table_hbm.at[idx_v.at[f]], rows[bf], sem_in[bf])

        def drain_gather(bf):
            pltpu.make_async_copy(
                table_hbm.at[pl.ds(0, BB)], rows[bf], sem_in[bf]).wait()

        def drain_store(bf):
            pltpu.make_async_copy(
                tbuf[bf], out_hbm.at[pl.ds(0, D_MODEL), pl.ds(0, BB)],
                sem_out[bf]).wait()

        for b in range(AHEAD):
            fire_gather(b, b)

        def group(i, carry):
            for b in range(NBUF):
                g = i * NBUF + b
                cid = cid0 + g
                s = cid // kb_n
                kb = cid - s * kb_n
                r = rows[b]
                t = tbuf[b]

                drain_gather(b)

                @pl.when(i > 0)
                def _drain(b=b):
                    drain_store(b)

                @plsc.parallel_loop(0, L, unroll=8)
                def diag(d, r=r, t=t):
                    rot = jnp.where(lanes < L - d, lanes + d, lanes + d - L)
                    for c0 in range(D_MODEL // L):
                        col = rot + (c0 * L)
                        for j0 in range(BB // L):
                            v = plsc.load_gather(r, [rvecs[j0], col])
                            plsc.store_scatter(t, [col, rvecs[j0]], v)

                pltpu.async_copy(
                    t,
                    out_hbm.at[pl.ds(s * D_MODEL, D_MODEL),
                               pl.ds(kb * BB, BB)],
                    sem_out[b])

                @pl.when(g + AHEAD < nch)
                def _fire(g=g, b=b):
                    fire_gather(g + AHEAD, (b + AHEAD) % NBUF)
            return carry

        lax.fori_loop(0, ngrp, group, 0)

        for b in range(NBUF):
            drain_store(b)

    return emb


@functools.lru_cache(maxsize=None)
def _get_emb(n_rows: int, n_cols: int):
    return _make_emb_kernel(n_rows, n_cols)


def kernel(x, table):
    n_rows, n_cols = x.shape
    idx = x.T.astype(jnp.int32).reshape(NW, (n_cols * n_rows) // (NW * BB), BB)
    tablep = jnp.pad(table * jnp.float32(SCALE), ((0, 0), (0, D_PAD - D_MODEL)))
    out = _get_emb(n_rows, n_cols)(idx, tablep)
    return out.reshape(n_cols, D_MODEL, n_rows).transpose(2, 0, 1)
